# baseline (speedup 1.0000x reference)
import functools
import jax
import jax.numpy as jnp
from jax import lax
from jax.experimental import pallas as pl
from jax.experimental.pallas import tpu as pltpu
from jax.experimental.pallas import tpu_sc as plsc

EMB = 64
PAIRW = 2 * EMB
SEQ = 200
NPAIR_L = SEQ // 2
CHUNK = 128
NUM_WORKERS = 32

_DNUMS = lax.GatherDimensionNumbers(
    offset_dims=(), collapsed_slice_dims=(0,), start_index_map=(0,))


def _shuffle(x, perm):
    return lax.gather(x, perm.reshape(16, 1), _DNUMS, slice_sizes=(1,),
                      mode=lax.GatherScatterMode.PROMISE_IN_BOUNDS)


def _sc_body(batch_hbm, table4_hbm, out_hbm, table_sh, tok_v, idx_v, rows_v,
             sem):
    n_pairs = out_hbm.shape[0]
    per_w = n_pairs // NUM_WORKERS
    n_chunks = per_w // CHUNK
    sid = lax.axis_index("s")
    wid = sid * 2 + lax.axis_index("c")
    base_w = wid * per_w

    @pl.when(sid == 0)
    def _():
        pltpu.sync_copy(table4_hbm, table_sh)
    plsc.subcore_barrier()

    iota16 = lax.iota(jnp.int32, 16)
    lane_lo = iota16 < 8
    perm_e_lo = jnp.where(lane_lo, 2 * iota16, 0)
    perm_e_hi = jnp.where(lane_lo, 0, 2 * iota16 - 16)
    perm_o_lo = perm_e_lo + jnp.where(lane_lo, 1, 0)
    perm_o_hi = perm_e_hi + jnp.where(lane_lo, 0, 1)

    def _chunk(c, _):
        base = base_w + c * CHUNK
        pltpu.sync_copy(batch_hbm.at[pl.ds(2 * base, 2 * CHUNK)], tok_v)

        def _idx(g, _):
            t0 = tok_v[pl.ds(32 * g, 16)]
            t1 = tok_v[pl.ds(32 * g + 16, 16)]
            e = jnp.where(lane_lo, _shuffle(t0, perm_e_lo),
                          _shuffle(t1, perm_e_hi))
            o = jnp.where(lane_lo, _shuffle(t0, perm_o_lo),
                          _shuffle(t1, perm_o_hi))
            m = (jnp.where(e != 0, 1, 0) + jnp.where(o != 0, 2, 0))
            lp = lax.rem(base + g * 16 + iota16, NPAIR_L)
            idx_v[pl.ds(g * 16, 16)] = lp * 4 + m
            return 0
        lax.fori_loop(0, CHUNK // 16, _idx, 0)

        pltpu.async_copy(table_sh.at[idx_v], rows_v, sem).wait()
        pltpu.sync_copy(rows_v, out_hbm.at[pl.ds(base, CHUNK)])
        return 0
    lax.fori_loop(0, n_chunks, _chunk, 0)


def _build_pair_table(emb_table):
    t = emb_table.at[0].set(0.0)
    left = t[1:SEQ:2]
    right = t[2:SEQ + 1:2]
    tb = jnp.zeros((NPAIR_L, 4, PAIRW), jnp.float32)
    tb = tb.at[:, 1, :EMB].set(left).at[:, 3, :EMB].set(left)
    tb = tb.at[:, 2, EMB:].set(right).at[:, 3, EMB:].set(right)
    return tb.reshape(NPAIR_L * 4, PAIRW)


def kernel(batch, emb_table):
    B, L = batch.shape
    E = emb_table.shape[1]
    n_pairs = B * L // 2
    batch_flat = batch.reshape(B * L)
    table4 = _build_pair_table(emb_table)
    mesh = plsc.VectorSubcoreMesh(core_axis_name="c", subcore_axis_name="s")
    out = pl.kernel(
        _sc_body,
        out_type=jax.ShapeDtypeStruct((n_pairs, PAIRW), jnp.float32),
        mesh=mesh,
        scratch_types=[
            pltpu.VMEM_SHARED((NPAIR_L * 4, PAIRW), jnp.float32),
            pltpu.VMEM((2 * CHUNK,), jnp.int32),
            pltpu.VMEM((CHUNK,), jnp.int32),
            pltpu.VMEM((CHUNK, PAIRW), jnp.float32),
            pltpu.SemaphoreType.DMA,
        ],
    )(batch_flat, table4)
    return out.reshape(B, L, E)


# speedup vs baseline: 1.1043x; 1.1043x over previous
import functools
import jax
import jax.numpy as jnp
from jax import lax
from jax.experimental import pallas as pl
from jax.experimental.pallas import tpu as pltpu
from jax.experimental.pallas import tpu_sc as plsc

EMB = 64
PAIRW = 2 * EMB
SEQ = 200
NPAIR_L = SEQ // 2
CHUNK = 128
NUM_WORKERS = 32

_DNUMS = lax.GatherDimensionNumbers(
    offset_dims=(), collapsed_slice_dims=(0,), start_index_map=(0,))


def _shuffle(x, perm):
    return lax.gather(x, perm.reshape(16, 1), _DNUMS, slice_sizes=(1,),
                      mode=lax.GatherScatterMode.PROMISE_IN_BOUNDS)


def _sc_body(batch_hbm, table4_hbm, out_hbm, table_sh, tok_v, idx_v,
             rows_v0, rows_v1, gsem, tsem, wsem0, wsem1):
    n_pairs = out_hbm.shape[0]
    per_w = n_pairs // NUM_WORKERS
    n_chunks = per_w // CHUNK
    n_iter = n_chunks // 2
    sid = lax.axis_index("s")
    wid = sid * 2 + lax.axis_index("c")
    base_w = wid * per_w

    @pl.when(sid == 0)
    def _():
        pltpu.sync_copy(table4_hbm, table_sh)
    plsc.subcore_barrier()

    iota16 = lax.iota(jnp.int32, 16)
    lane_lo = iota16 < 8
    perm_e_lo = jnp.where(lane_lo, 2 * iota16, 0)
    perm_e_hi = jnp.where(lane_lo, 0, 2 * iota16 - 16)
    perm_o_lo = perm_e_lo + jnp.where(lane_lo, 1, 0)
    perm_o_hi = perm_e_hi + jnp.where(lane_lo, 0, 1)

    def _tok_start(c, half):
        pltpu.async_copy(
            batch_hbm.at[pl.ds(2 * (base_w + c * CHUNK), 2 * CHUNK)],
            tok_v.at[half], tsem)

    def _tok_wait(half):
        pltpu.make_async_copy(
            batch_hbm.at[pl.ds(0, 2 * CHUNK)], tok_v.at[half], tsem).wait()

    def _compute_idx(c, half):
        base = base_w + c * CHUNK

        def _idx(g, _):
            t0 = tok_v[half, pl.ds(32 * g, 16)]
            t1 = tok_v[half, pl.ds(32 * g + 16, 16)]
            e = jnp.where(lane_lo, _shuffle(t0, perm_e_lo),
                          _shuffle(t1, perm_e_hi))
            o = jnp.where(lane_lo, _shuffle(t0, perm_o_lo),
                          _shuffle(t1, perm_o_hi))
            m = (jnp.where(e != 0, 1, 0) + jnp.where(o != 0, 2, 0))
            lp = lax.rem(base + g * 16 + iota16, NPAIR_L)
            idx_v[half, pl.ds(g * 16, 16)] = lp * 4 + m
            return 0
        lax.fori_loop(0, CHUNK // 16, _idx, 0)

    def _write_wait(rows_v, wsem):
        pltpu.make_async_copy(
            rows_v, out_hbm.at[pl.ds(base_w, CHUNK)], wsem).wait()

    def _do_chunk(i, c, half, rows_v, wsem):
        _tok_wait(half)
        _compute_idx(c, half)
        @pl.when(c + 2 < n_chunks)
        def _():
            _tok_start(c + 2, half)
        @pl.when(i > 0)
        def _():
            _write_wait(rows_v, wsem)
        pltpu.async_copy(table_sh.at[idx_v.at[half]], rows_v, gsem).wait()
        pltpu.async_copy(rows_v, out_hbm.at[pl.ds(base_w + c * CHUNK, CHUNK)],
                         wsem)

    _tok_start(0, 0)
    _tok_start(1, 1)

    def _iter(i, _):
        _do_chunk(i, 2 * i, 0, rows_v0, wsem0)
        _do_chunk(i, 2 * i + 1, 1, rows_v1, wsem1)
        return 0
    lax.fori_loop(0, n_iter, _iter, 0)

    _write_wait(rows_v0, wsem0)
    _write_wait(rows_v1, wsem1)


def _build_pair_table(emb_table):
    t = emb_table.at[0].set(0.0)
    left = t[1:SEQ:2]
    right = t[2:SEQ + 1:2]
    tb = jnp.zeros((NPAIR_L, 4, PAIRW), jnp.float32)
    tb = tb.at[:, 1, :EMB].set(left).at[:, 3, :EMB].set(left)
    tb = tb.at[:, 2, EMB:].set(right).at[:, 3, EMB:].set(right)
    return tb.reshape(NPAIR_L * 4, PAIRW)


def kernel(batch, emb_table):
    B, L = batch.shape
    E = emb_table.shape[1]
    n_pairs = B * L // 2
    batch_flat = batch.reshape(B * L)
    table4 = _build_pair_table(emb_table)
    mesh = plsc.VectorSubcoreMesh(core_axis_name="c", subcore_axis_name="s")
    out = pl.kernel(
        _sc_body,
        out_type=jax.ShapeDtypeStruct((n_pairs, PAIRW), jnp.float32),
        mesh=mesh,
        scratch_types=[
            pltpu.VMEM_SHARED((NPAIR_L * 4, PAIRW), jnp.float32),
            pltpu.VMEM((2, 2 * CHUNK), jnp.int32),
            pltpu.VMEM((2, CHUNK), jnp.int32),
            pltpu.VMEM((CHUNK, PAIRW), jnp.float32),
            pltpu.VMEM((CHUNK, PAIRW), jnp.float32),
            pltpu.SemaphoreType.DMA,
            pltpu.SemaphoreType.DMA,
            pltpu.SemaphoreType.DMA,
            pltpu.SemaphoreType.DMA,
        ],
    )(batch_flat, table4)
    return out.reshape(B, L, E)
